# baseline (speedup 1.0000x reference)
import functools

import jax
import jax.numpy as jnp
import numpy as np
from jax import lax
from jax.experimental import pallas as pl
from jax.experimental.pallas import tpu as pltpu
from jax.experimental.pallas import tpu_sc as plsc

_TOPK = 16
_KB = 2048
_GS = 64
_CAP = 64
_INT_MAX = 2**31 - 1
_F32_INF = np.float32(np.inf)


def _dist_tile_body(q_ref, k_ref, d_out, g_out, *, kb, k_real):
    q = q_ref[...]
    kblk = k_ref[...]
    qk = lax.dot_general(q, kblk, (((1,), (1,)), ((), ())),
                         preferred_element_type=jnp.float32)
    q_sq = jnp.sum(q * q, axis=1, keepdims=True)
    k_sq = jnp.sum(kblk * kblk, axis=1)
    d2 = q_sq + k_sq[None, :] - 2.0 * qk
    col0 = pl.program_id(0) * kb
    cols = col0 + lax.broadcasted_iota(jnp.int32, (1, kb), 1)
    d2 = jnp.where(cols >= k_real, _F32_INF, d2)
    slices = [d2[:, g * _GS:(g + 1) * _GS] for g in range(kb // _GS)]
    d_out[...] = jnp.concatenate(slices, axis=0)
    gm = jnp.concatenate(
        [jnp.min(s, axis=1, keepdims=True) for s in slices], axis=1)
    g_out[...] = gm[None]


def _dist_tiles(queries, keys_p, k_real):
    q_n, d_n = queries.shape
    kp = keys_p.shape[0]
    nkb = kp // _KB
    ng = _KB // _GS
    body = functools.partial(_dist_tile_body, kb=_KB, k_real=k_real)
    return pl.pallas_call(
        body,
        grid=(nkb,),
        in_specs=[
            pl.BlockSpec((q_n, d_n), lambda i: (0, 0)),
            pl.BlockSpec((_KB, d_n), lambda i: (i, 0)),
        ],
        out_specs=[
            pl.BlockSpec((q_n * ng, _GS), lambda i: (i, 0)),
            pl.BlockSpec((1, q_n, ng), lambda i: (i, 0, 0)),
        ],
        out_shape=[
            jax.ShapeDtypeStruct((q_n * ng * nkb, _GS), jnp.float32),
            jax.ShapeDtypeStruct((nkb, q_n, ng), jnp.float32),
        ],
    )(queries, keys_p)


def _gsel_body(g_ref, o_ref, *, topk, ng_real):
    d = g_ref[...]
    w = d.shape[1]
    cols = lax.broadcasted_iota(jnp.int32, (1, w), 1)
    d = jnp.where(cols >= ng_real, _F32_INF, d)
    outs = []
    for _ in range(topk):
        m = jnp.min(d, axis=1, keepdims=True)
        eq = d == m
        sel = jnp.min(jnp.where(eq, cols, _INT_MAX), axis=1, keepdims=True)
        outs.append(sel)
        d = jnp.where(cols == sel, _F32_INF, d)
    o_ref[...] = jnp.concatenate(outs, axis=1)


def _select_groups(gmins, ng_real):
    q_n, ngp = gmins.shape
    body = functools.partial(_gsel_body, topk=_TOPK, ng_real=ng_real)
    return pl.pallas_call(
        body,
        in_specs=[pl.BlockSpec((q_n, ngp), lambda: (0, 0))],
        out_specs=pl.BlockSpec((q_n, _TOPK), lambda: (0, 0)),
        out_shape=jax.ShapeDtypeStruct((q_n, _TOPK), jnp.int32),
    )(gmins)


def _lane_bcast(v, j):
    return v.at[jnp.full((16,), j, jnp.int32)].get(mode="promise_in_bounds")


def _sc_select_gather(gsel, d2flat, key_feats, keys, ng_total, q_n):
    info = plsc.get_sparse_core_info()
    nc = info.num_cores
    nw = nc * info.num_subcores
    d_n = key_feats.shape[1]
    qpw = q_n // nw
    spg = _GS // 16
    nch = (qpw * _TOPK) // 128
    qpc = 128 // _TOPK
    mesh = plsc.VectorSubcoreMesh(core_axis_name="c", subcore_axis_name="s")

    @functools.partial(
        pl.kernel,
        mesh=mesh,
        out_type=[
            jax.ShapeDtypeStruct((q_n * _TOPK, d_n), jnp.float32),
            jax.ShapeDtypeStruct((q_n * _TOPK, d_n), jnp.float32),
        ],
        scratch_types=[
            pltpu.VMEM((qpw, _TOPK), jnp.int32),
            pltpu.VMEM((nch, 128), jnp.int32),
            pltpu.VMEM((nch, 128, _GS), jnp.float32),
            pltpu.VMEM((nch, 128), jnp.int32),
            pltpu.VMEM((nch, 128), jnp.int32),
            pltpu.VMEM((_CAP,), jnp.float32),
            pltpu.VMEM((_CAP,), jnp.int32),
            pltpu.VMEM((128, d_n), jnp.float32),
            pltpu.VMEM((128, d_n), jnp.float32),
            pltpu.SemaphoreType.DMA,
            pltpu.SemaphoreType.DMA,
        ],
        compiler_params=pltpu.CompilerParams(use_tc_tiling_on_sc=False,
                                             needs_layout_passes=False),
    )
    def body(gsel_hbm, d2_hbm, feats_hbm, keys_hbm, out_f, out_k,
             gsel_v, fidx, dbuf, gbuf, sbuf, cand_d, cand_i, rows_f, rows_k,
             s1, s2):
        wid = lax.axis_index("s") * nc + lax.axis_index("c")
        qbase = wid * qpw
        iota16 = lax.iota(jnp.int32, 16)

        pltpu.sync_copy(gsel_hbm.at[pl.ds(qbase, qpw)], gsel_v)

        def build(ql, _):
            c = ql // qpc
            r0 = (ql % qpc) * _TOPK
            g16 = gsel_v[ql]
            fidx[c, pl.ds(r0, _TOPK)] = g16 * q_n + (qbase + ql)
            sbuf[c, pl.ds(r0, _TOPK)] = iota16 * q_n + (qbase + ql)
            return 0
        lax.fori_loop(0, qpw, build, 0)

        copies = [pltpu.async_copy(d2_hbm.at[fidx.at[c]], dbuf.at[c], s1)
                  for c in range(nch)]
        for cp in copies:
            cp.wait()

        def per_query(ql, _):
            c = ql // qpc
            r0 = (ql % qpc) * _TOPK
            g16 = gsel_v[ql]

            def p1(t, r):
                v = dbuf[c, r0 + t // spg, pl.ds((t % spg) * 16, 16)]
                sv = lax.sort(v)
                return lax.sort(jnp.minimum(r, lax.rev(sv, (0,))))
            r = lax.fori_loop(0, _TOPK * spg, p1, jnp.full((16,), _F32_INF))
            v16 = _lane_bcast(r, 15)

            for b in range(_CAP // 16):
                cand_d[pl.ds(b * 16, 16)] = jnp.full((16,), _F32_INF)
                cand_i[pl.ds(b * 16, 16)] = jnp.full((16,), _INT_MAX, jnp.int32)

            def p2(t, cnt):
                j = t // spg
                s = t % spg
                v = dbuf[c, r0 + j, pl.ds(s * 16, 16)]
                gj = _lane_bcast(g16, j)
                iv = gj * _GS + s * 16 + iota16
                m = (v <= v16) & (cnt < _CAP - 16)
                plsc.store_compressed(cand_d.at[pl.ds(cnt, 16)], v, mask=m)
                plsc.store_compressed(cand_i.at[pl.ds(cnt, 16)], iv, mask=m)
                npick = jnp.max(plsc.all_reduce_population_count(m))
                return cnt + npick
            lax.fori_loop(0, _TOPK * spg, p2, jnp.int32(0))

            def p3(it, carry):
                out_i, cd0, cd1, cd2, cd3, ci0, ci1, ci2, ci3 = carry
                cds = (cd0, cd1, cd2, cd3)
                cis = (ci0, ci1, ci2, ci3)
                mv = cds[0]
                for x in cds[1:]:
                    mv = jnp.minimum(mv, x)
                ms = jnp.min(mv)
                msv = jnp.full((16,), ms)
                iv = jnp.full((16,), _INT_MAX, jnp.int32)
                for x, y in zip(cds, cis):
                    iv = jnp.minimum(iv, jnp.where(x == msv, y, _INT_MAX))
                isv = jnp.full((16,), jnp.min(iv))
                out_i = jnp.where(iota16 == it, isv, out_i)
                new_cds = tuple(
                    jnp.where((x == msv) & (y == isv), _F32_INF, x)
                    for x, y in zip(cds, cis))
                return (out_i,) + new_cds + cis
            init = (jnp.full((16,), 0, jnp.int32),
                    cand_d[pl.ds(0, 16)], cand_d[pl.ds(16, 16)],
                    cand_d[pl.ds(32, 16)], cand_d[pl.ds(48, 16)],
                    cand_i[pl.ds(0, 16)], cand_i[pl.ds(16, 16)],
                    cand_i[pl.ds(32, 16)], cand_i[pl.ds(48, 16)])
            out_i = lax.fori_loop(0, _TOPK, p3, init)[0]
            gbuf[c, pl.ds(r0, _TOPK)] = out_i
            return 0
        lax.fori_loop(0, qpw, per_query, 0)

        for c in range(nch):
            cf = pltpu.async_copy(feats_hbm.at[gbuf.at[c]], rows_f, s1)
            ck = pltpu.async_copy(keys_hbm.at[gbuf.at[c]], rows_k, s2)
            cf.wait()
            ck.wait()
            sf = pltpu.async_copy(rows_f, out_f.at[sbuf.at[c]], s1)
            sk = pltpu.async_copy(rows_k, out_k.at[sbuf.at[c]], s2)
            sf.wait()
            sk.wait()

    return body(gsel, d2flat, key_feats, keys)


def _head_body(fj_ref, kn_ref, q_ref, wp_ref, w_ref, o_ref):
    j = pl.program_id(1)
    dp = q_ref[...] - kn_ref[...]
    pe = jnp.dot(dp, wp_ref[...], preferred_element_type=jnp.float32)
    f = jnp.maximum(
        jnp.dot(fj_ref[...] + pe, w_ref[...], preferred_element_type=jnp.float32),
        0.0)

    @pl.when(j == 0)
    def _():
        o_ref[...] = f

    @pl.when(j > 0)
    def _():
        o_ref[...] = jnp.maximum(o_ref[...], f)


def _head(fj_nm, kn_nm, queries, w_pos, w):
    q_n, d_n = queries.shape
    h_n = w.shape[1]
    qb = 128
    nqb = q_n // qb
    return pl.pallas_call(
        _head_body,
        grid=(nqb, _TOPK),
        in_specs=[
            pl.BlockSpec((qb, d_n), lambda i, j: (j * nqb + i, 0)),
            pl.BlockSpec((qb, d_n), lambda i, j: (j * nqb + i, 0)),
            pl.BlockSpec((qb, d_n), lambda i, j: (i, 0)),
            pl.BlockSpec((d_n, d_n), lambda i, j: (0, 0)),
            pl.BlockSpec((d_n, h_n), lambda i, j: (0, 0)),
        ],
        out_specs=pl.BlockSpec((qb, h_n), lambda i, j: (i, 0)),
        out_shape=jax.ShapeDtypeStruct((q_n, h_n), jnp.float32),
    )(fj_nm, kn_nm, queries, w_pos, w)


def kernel(queries, keys, key_feats, W_pos, W, k):
    q_n, d_n = queries.shape
    k_n = keys.shape[0]
    nkb = -(-k_n // _KB)
    kp = nkb * _KB
    keys_p = jnp.pad(keys, ((0, kp - k_n), (0, 0)))

    d2flat, gm3 = _dist_tiles(queries, keys_p, k_n)
    ng = kp // _GS
    gmins = jnp.transpose(gm3, (1, 0, 2)).reshape(q_n, ng)
    ngp = -(-ng // 128) * 128
    gmins = jnp.pad(gmins, ((0, 0), (0, ngp - ng)), constant_values=jnp.inf)
    gsel = _select_groups(gmins, ng)

    fj_nm, kn_nm = _sc_select_gather(gsel, d2flat, key_feats, keys, ng, q_n)

    return _head(fj_nm, kn_nm, queries, W_pos, W)


# speedup vs baseline: 2.0063x; 2.0063x over previous
import functools

import jax
import jax.numpy as jnp
import numpy as np
from jax import lax
from jax.experimental import pallas as pl
from jax.experimental.pallas import tpu as pltpu
from jax.experimental.pallas import tpu_sc as plsc

_TOPK = 16
_KB = 2048
_GS = 64
_CAP = 64
_INT_MAX = 2**31 - 1
_F32_INF = np.float32(np.inf)


def _dist_tile_body(q_ref, k_ref, d_out, g_out, *, kb, k_real):
    q = q_ref[...]
    kblk = k_ref[...]
    qk = lax.dot_general(q, kblk, (((1,), (1,)), ((), ())),
                         preferred_element_type=jnp.float32)
    q_sq = jnp.sum(q * q, axis=1, keepdims=True)
    k_sq = jnp.sum(kblk * kblk, axis=1)
    d2 = q_sq + k_sq[None, :] - 2.0 * qk
    col0 = pl.program_id(0) * kb
    cols = col0 + lax.broadcasted_iota(jnp.int32, (1, kb), 1)
    d2 = jnp.where(cols >= k_real, _F32_INF, d2)
    d_out[...] = jnp.concatenate(
        [d2[:, s * 128:(s + 1) * 128] for s in range(kb // 128)], axis=0)
    gm = jnp.concatenate(
        [jnp.min(d2[:, g * _GS:(g + 1) * _GS], axis=1, keepdims=True)
         for g in range(kb // _GS)], axis=1)
    g_out[...] = gm[None]


def _dist_tiles(queries, keys_p, k_real):
    q_n, d_n = queries.shape
    kp = keys_p.shape[0]
    nkb = kp // _KB
    ng = _KB // _GS
    body = functools.partial(_dist_tile_body, kb=_KB, k_real=k_real)
    return pl.pallas_call(
        body,
        grid=(nkb,),
        in_specs=[
            pl.BlockSpec((q_n, d_n), lambda i: (0, 0)),
            pl.BlockSpec((_KB, d_n), lambda i: (i, 0)),
        ],
        out_specs=[
            pl.BlockSpec((q_n * (_KB // 128), 128), lambda i: (i, 0)),
            pl.BlockSpec((1, q_n, ng), lambda i: (i, 0, 0)),
        ],
        out_shape=[
            jax.ShapeDtypeStruct((q_n * (kp // 128), 128), jnp.float32),
            jax.ShapeDtypeStruct((nkb, q_n, ng), jnp.float32),
        ],
    )(queries, keys_p)


def _gsel_body(g_ref, o_ref, *, topk, ng_real):
    d = g_ref[...]
    w = d.shape[1]
    cols = lax.broadcasted_iota(jnp.int32, (1, w), 1)
    d = jnp.where(cols >= ng_real, _F32_INF, d)
    outs = []
    for _ in range(topk):
        m = jnp.min(d, axis=1, keepdims=True)
        eq = d == m
        sel = jnp.min(jnp.where(eq, cols, _INT_MAX), axis=1, keepdims=True)
        outs.append(sel)
        d = jnp.where(cols == sel, _F32_INF, d)
    o_ref[...] = jnp.concatenate(outs, axis=1)


def _select_groups(gmins, ng_real):
    q_n, ngp = gmins.shape
    body = functools.partial(_gsel_body, topk=_TOPK, ng_real=ng_real)
    return pl.pallas_call(
        body,
        in_specs=[pl.BlockSpec((q_n, ngp), lambda: (0, 0))],
        out_specs=pl.BlockSpec((q_n, _TOPK), lambda: (0, 0)),
        out_shape=jax.ShapeDtypeStruct((q_n, _TOPK), jnp.int32),
    )(gmins)


def _lane_bcast(v, j):
    return v.at[jnp.full((16,), j, jnp.int32)].get(mode="promise_in_bounds")


def _sc_select_gather(gsel, d2flat, key_feats, keys, ng_total, q_n):
    info = plsc.get_sparse_core_info()
    nc = info.num_cores
    nw = nc * info.num_subcores
    d_n = key_feats.shape[1]
    qpw = q_n // nw
    spg = _GS // 16
    nch = (qpw * _TOPK) // 128
    qpc = 128 // _TOPK
    mesh = plsc.VectorSubcoreMesh(core_axis_name="c", subcore_axis_name="s")

    @functools.partial(
        pl.kernel,
        mesh=mesh,
        out_type=[
            jax.ShapeDtypeStruct((q_n * _TOPK, d_n), jnp.float32),
            jax.ShapeDtypeStruct((q_n * _TOPK, d_n), jnp.float32),
        ],
        scratch_types=[
            pltpu.VMEM((qpw, _TOPK), jnp.int32),
            pltpu.VMEM((nch, 128), jnp.int32),
            pltpu.VMEM((nch, 128, 128), jnp.float32),
            pltpu.VMEM((nch, 128), jnp.int32),
            pltpu.VMEM((nch, 128), jnp.int32),
            pltpu.VMEM((_CAP,), jnp.float32),
            pltpu.VMEM((_CAP,), jnp.int32),
            pltpu.VMEM((128, d_n), jnp.float32),
            pltpu.VMEM((128, d_n), jnp.float32),
            pltpu.SemaphoreType.DMA,
            pltpu.SemaphoreType.DMA,
        ],
        compiler_params=pltpu.CompilerParams(use_tc_tiling_on_sc=False,
                                             needs_layout_passes=False),
    )
    def body(gsel_hbm, d2_hbm, feats_hbm, keys_hbm, out_f, out_k,
             gsel_v, fidx, dbuf, gbuf, sbuf, cand_d, cand_i, rows_f, rows_k,
             s1, s2):
        wid = lax.axis_index("s") * nc + lax.axis_index("c")
        qbase = wid * qpw
        iota16 = lax.iota(jnp.int32, 16)

        pltpu.sync_copy(gsel_hbm.at[pl.ds(qbase, qpw)], gsel_v)

        def build(ql, _):
            c = ql // qpc
            r0 = (ql % qpc) * _TOPK
            g16 = gsel_v[ql]
            fidx[c, pl.ds(r0, _TOPK)] = (g16 >> 1) * q_n + (qbase + ql)
            sbuf[c, pl.ds(r0, _TOPK)] = iota16 * q_n + (qbase + ql)
            return 0
        lax.fori_loop(0, qpw, build, 0)

        copies = [pltpu.async_copy(d2_hbm.at[fidx.at[c]], dbuf.at[c], s1)
                  for c in range(nch)]
        for cp in copies:
            cp.wait()

        def per_query(ql, _):
            c = ql // qpc
            r0 = (ql % qpc) * _TOPK
            g16 = gsel_v[ql]

            def p1(t, r):
                j = t // spg
                h = jnp.max(_lane_bcast(g16, j)) & 1
                v = dbuf[c, r0 + j, pl.ds(h * _GS + (t % spg) * 16, 16)]
                sv = lax.sort(v)
                return lax.sort(jnp.minimum(r, lax.rev(sv, (0,))))
            r = lax.fori_loop(0, _TOPK * spg, p1, jnp.full((16,), _F32_INF))
            v16 = _lane_bcast(r, 15)

            for b in range(_CAP // 16):
                cand_d[pl.ds(b * 16, 16)] = jnp.full((16,), _F32_INF)
                cand_i[pl.ds(b * 16, 16)] = jnp.full((16,), _INT_MAX, jnp.int32)

            def p2(t, cnt):
                j = t // spg
                s = t % spg
                gj = _lane_bcast(g16, j)
                h = jnp.max(gj) & 1
                v = dbuf[c, r0 + j, pl.ds(h * _GS + s * 16, 16)]
                iv = gj * _GS + s * 16 + iota16
                m = (v <= v16) & (cnt < _CAP - 16)
                plsc.store_compressed(cand_d.at[pl.ds(cnt, 16)], v, mask=m)
                plsc.store_compressed(cand_i.at[pl.ds(cnt, 16)], iv, mask=m)
                npick = jnp.max(plsc.all_reduce_population_count(m))
                return cnt + npick
            lax.fori_loop(0, _TOPK * spg, p2, jnp.int32(0))

            def p3(it, carry):
                out_i, cd0, cd1, cd2, cd3, ci0, ci1, ci2, ci3 = carry
                cds = (cd0, cd1, cd2, cd3)
                cis = (ci0, ci1, ci2, ci3)
                mv = cds[0]
                for x in cds[1:]:
                    mv = jnp.minimum(mv, x)
                ms = jnp.min(mv)
                msv = jnp.full((16,), ms)
                iv = jnp.full((16,), _INT_MAX, jnp.int32)
                for x, y in zip(cds, cis):
                    iv = jnp.minimum(iv, jnp.where(x == msv, y, _INT_MAX))
                isv = jnp.full((16,), jnp.min(iv))
                out_i = jnp.where(iota16 == it, isv, out_i)
                new_cds = tuple(
                    jnp.where((x == msv) & (y == isv), _F32_INF, x)
                    for x, y in zip(cds, cis))
                return (out_i,) + new_cds + cis
            init = (jnp.full((16,), 0, jnp.int32),
                    cand_d[pl.ds(0, 16)], cand_d[pl.ds(16, 16)],
                    cand_d[pl.ds(32, 16)], cand_d[pl.ds(48, 16)],
                    cand_i[pl.ds(0, 16)], cand_i[pl.ds(16, 16)],
                    cand_i[pl.ds(32, 16)], cand_i[pl.ds(48, 16)])
            out_i = lax.fori_loop(0, _TOPK, p3, init)[0]
            gbuf[c, pl.ds(r0, _TOPK)] = out_i
            return 0
        lax.fori_loop(0, qpw, per_query, 0)

        for c in range(nch):
            cf = pltpu.async_copy(feats_hbm.at[gbuf.at[c]], rows_f, s1)
            ck = pltpu.async_copy(keys_hbm.at[gbuf.at[c]], rows_k, s2)
            cf.wait()
            ck.wait()
            sf = pltpu.async_copy(rows_f, out_f.at[sbuf.at[c]], s1)
            sk = pltpu.async_copy(rows_k, out_k.at[sbuf.at[c]], s2)
            sf.wait()
            sk.wait()

    return body(gsel, d2flat, key_feats, keys)


def _head_body(fj_ref, kn_ref, q_ref, wp_ref, w_ref, o_ref):
    j = pl.program_id(1)
    dp = q_ref[...] - kn_ref[...]
    pe = jnp.dot(dp, wp_ref[...], preferred_element_type=jnp.float32)
    f = jnp.maximum(
        jnp.dot(fj_ref[...] + pe, w_ref[...], preferred_element_type=jnp.float32),
        0.0)

    @pl.when(j == 0)
    def _():
        o_ref[...] = f

    @pl.when(j > 0)
    def _():
        o_ref[...] = jnp.maximum(o_ref[...], f)


def _head(fj_nm, kn_nm, queries, w_pos, w):
    q_n, d_n = queries.shape
    h_n = w.shape[1]
    qb = 128
    nqb = q_n // qb
    return pl.pallas_call(
        _head_body,
        grid=(nqb, _TOPK),
        in_specs=[
            pl.BlockSpec((qb, d_n), lambda i, j: (j * nqb + i, 0)),
            pl.BlockSpec((qb, d_n), lambda i, j: (j * nqb + i, 0)),
            pl.BlockSpec((qb, d_n), lambda i, j: (i, 0)),
            pl.BlockSpec((d_n, d_n), lambda i, j: (0, 0)),
            pl.BlockSpec((d_n, h_n), lambda i, j: (0, 0)),
        ],
        out_specs=pl.BlockSpec((qb, h_n), lambda i, j: (i, 0)),
        out_shape=jax.ShapeDtypeStruct((q_n, h_n), jnp.float32),
    )(fj_nm, kn_nm, queries, w_pos, w)


def kernel(queries, keys, key_feats, W_pos, W, k):
    q_n, d_n = queries.shape
    k_n = keys.shape[0]
    nkb = -(-k_n // _KB)
    kp = nkb * _KB
    keys_p = jnp.pad(keys, ((0, kp - k_n), (0, 0)))

    d2flat, gm3 = _dist_tiles(queries, keys_p, k_n)
    ng = kp // _GS
    gmins = jnp.transpose(gm3, (1, 0, 2)).reshape(q_n, ng)
    ngp = -(-ng // 128) * 128
    gmins = jnp.pad(gmins, ((0, 0), (0, ngp - ng)), constant_values=jnp.inf)
    gsel = _select_groups(gmins, ng)

    fj_nm, kn_nm = _sc_select_gather(gsel, d2flat, key_feats, keys, ng, q_n)

    return _head(fj_nm, kn_nm, queries, W_pos, W)


# speedup vs baseline: 2.0617x; 1.0276x over previous
import functools

import jax
import jax.numpy as jnp
import numpy as np
from jax import lax
from jax.experimental import pallas as pl
from jax.experimental.pallas import tpu as pltpu
from jax.experimental.pallas import tpu_sc as plsc

_TOPK = 16
_KB = 2048
_GS = 64
_CAP = 64
_INT_MAX = 2**31 - 1
_F32_INF = np.float32(np.inf)


def _dist_tile_body(q_ref, k_ref, d_out, g_out, *, kb, k_real):
    q = q_ref[...]
    kblk = k_ref[...]
    qk = lax.dot_general(q, kblk, (((1,), (1,)), ((), ())),
                         preferred_element_type=jnp.float32)
    q_sq = jnp.sum(q * q, axis=1, keepdims=True)
    k_sq = jnp.sum(kblk * kblk, axis=1)
    d2 = q_sq + k_sq[None, :] - 2.0 * qk
    col0 = pl.program_id(0) * kb
    cols = col0 + lax.broadcasted_iota(jnp.int32, (1, kb), 1)
    d2 = jnp.where(cols >= k_real, _F32_INF, d2)
    d_out[...] = jnp.concatenate(
        [d2[:, s * 128:(s + 1) * 128] for s in range(kb // 128)], axis=0)
    gm = jnp.concatenate(
        [jnp.min(d2[:, g * _GS:(g + 1) * _GS], axis=1, keepdims=True)
         for g in range(kb // _GS)], axis=1)
    g_out[...] = gm[None]


def _dist_tiles(queries, keys, k_real):
    q_n, d_n = queries.shape
    nkb = -(-k_real // _KB)
    kp = nkb * _KB
    ng = _KB // _GS
    body = functools.partial(_dist_tile_body, kb=_KB, k_real=k_real)
    return pl.pallas_call(
        body,
        grid=(nkb,),
        in_specs=[
            pl.BlockSpec((q_n, d_n), lambda i: (0, 0)),
            pl.BlockSpec((_KB, d_n), lambda i: (i, 0)),
        ],
        out_specs=[
            pl.BlockSpec((q_n * (_KB // 128), 128), lambda i: (i, 0)),
            pl.BlockSpec((1, q_n, ng), lambda i: (i, 0, 0)),
        ],
        out_shape=[
            jax.ShapeDtypeStruct((q_n * (kp // 128), 128), jnp.float32),
            jax.ShapeDtypeStruct((nkb, q_n, ng), jnp.float32),
        ],
    )(queries, keys)


def _gsel_body(g_ref, o_ref, *, topk, ng_real):
    d = g_ref[...]
    w = d.shape[1]
    cols = lax.broadcasted_iota(jnp.int32, (1, w), 1)
    d = jnp.where(cols >= ng_real, _F32_INF, d)
    outs = []
    for _ in range(topk):
        m = jnp.min(d, axis=1, keepdims=True)
        eq = d == m
        sel = jnp.min(jnp.where(eq, cols, _INT_MAX), axis=1, keepdims=True)
        outs.append(sel)
        d = jnp.where(cols == sel, _F32_INF, d)
    o_ref[...] = jnp.concatenate(outs, axis=1)


def _select_groups(gmins, ng_real):
    q_n, ngp = gmins.shape
    body = functools.partial(_gsel_body, topk=_TOPK, ng_real=ng_real)
    return pl.pallas_call(
        body,
        in_specs=[pl.BlockSpec((q_n, ngp), lambda: (0, 0))],
        out_specs=pl.BlockSpec((q_n, _TOPK), lambda: (0, 0)),
        out_shape=jax.ShapeDtypeStruct((q_n, _TOPK), jnp.int32),
    )(gmins)


def _lane_bcast(v, j):
    return v.at[jnp.full((16,), j, jnp.int32)].get(mode="promise_in_bounds")


def _sc_select_gather(gsel, d2flat, key_feats, keys, ng_total, q_n):
    info = plsc.get_sparse_core_info()
    nc = info.num_cores
    nw = nc * info.num_subcores
    d_n = key_feats.shape[1]
    qpw = q_n // nw
    spg = _GS // 16
    nch = (qpw * _TOPK) // 128
    qpc = 128 // _TOPK
    mesh = plsc.VectorSubcoreMesh(core_axis_name="c", subcore_axis_name="s")

    @functools.partial(
        pl.kernel,
        mesh=mesh,
        out_type=[
            jax.ShapeDtypeStruct((q_n * _TOPK, d_n), jnp.float32),
            jax.ShapeDtypeStruct((q_n * _TOPK, d_n), jnp.float32),
        ],
        scratch_types=[
            pltpu.VMEM((qpw, _TOPK), jnp.int32),
            pltpu.VMEM((nch, 128), jnp.int32),
            pltpu.VMEM((nch, 128, 128), jnp.float32),
            pltpu.VMEM((nch, 128), jnp.int32),
            pltpu.VMEM((nch, 128), jnp.int32),
            pltpu.VMEM((_CAP,), jnp.float32),
            pltpu.VMEM((_CAP,), jnp.int32),
            pltpu.VMEM((128, d_n), jnp.float32),
            pltpu.VMEM((128, d_n), jnp.float32),
            pltpu.SemaphoreType.DMA,
            pltpu.SemaphoreType.DMA,
        ],
        compiler_params=pltpu.CompilerParams(use_tc_tiling_on_sc=False,
                                             needs_layout_passes=False),
    )
    def body(gsel_hbm, d2_hbm, feats_hbm, keys_hbm, out_f, out_k,
             gsel_v, fidx, dbuf, gbuf, sbuf, cand_d, cand_i, rows_f, rows_k,
             s1, s2):
        wid = lax.axis_index("s") * nc + lax.axis_index("c")
        qbase = wid * qpw
        iota16 = lax.iota(jnp.int32, 16)

        pltpu.sync_copy(gsel_hbm.at[pl.ds(qbase, qpw)], gsel_v)

        def build(ql, _):
            c = ql // qpc
            r0 = (ql % qpc) * _TOPK
            g16 = gsel_v[ql]
            fidx[c, pl.ds(r0, _TOPK)] = (g16 >> 1) * q_n + (qbase + ql)
            sbuf[c, pl.ds(r0, _TOPK)] = iota16 * q_n + (qbase + ql)
            return 0
        lax.fori_loop(0, qpw, build, 0)

        copies = [pltpu.async_copy(d2_hbm.at[fidx.at[c]], dbuf.at[c], s1)
                  for c in range(nch)]
        for cp in copies:
            cp.wait()

        def per_query(ql, _):
            c = ql // qpc
            r0 = (ql % qpc) * _TOPK
            g16 = gsel_v[ql]

            def p1(t, r):
                j = t // spg
                h = jnp.max(_lane_bcast(g16, j)) & 1
                v = dbuf[c, r0 + j, pl.ds(h * _GS + (t % spg) * 16, 16)]
                sv = lax.sort(v)
                return lax.sort(jnp.minimum(r, lax.rev(sv, (0,))))
            r = lax.fori_loop(0, _TOPK * spg, p1, jnp.full((16,), _F32_INF))
            v16 = _lane_bcast(r, 15)

            for b in range(_CAP // 16):
                cand_d[pl.ds(b * 16, 16)] = jnp.full((16,), _F32_INF)
                cand_i[pl.ds(b * 16, 16)] = jnp.full((16,), _INT_MAX, jnp.int32)

            def p2(t, cnt):
                j = t // spg
                s = t % spg
                gj = _lane_bcast(g16, j)
                h = jnp.max(gj) & 1
                v = dbuf[c, r0 + j, pl.ds(h * _GS + s * 16, 16)]
                iv = gj * _GS + s * 16 + iota16
                m = (v <= v16) & (cnt < _CAP - 16)
                plsc.store_compressed(cand_d.at[pl.ds(cnt, 16)], v, mask=m)
                plsc.store_compressed(cand_i.at[pl.ds(cnt, 16)], iv, mask=m)
                npick = jnp.max(plsc.all_reduce_population_count(m))
                return cnt + npick
            lax.fori_loop(0, _TOPK * spg, p2, jnp.int32(0))

            def p3(it, carry):
                out_i, cd0, cd1, cd2, cd3, ci0, ci1, ci2, ci3 = carry
                cds = (cd0, cd1, cd2, cd3)
                cis = (ci0, ci1, ci2, ci3)
                mv = cds[0]
                for x in cds[1:]:
                    mv = jnp.minimum(mv, x)
                ms = jnp.min(mv)
                msv = jnp.full((16,), ms)
                iv = jnp.full((16,), _INT_MAX, jnp.int32)
                for x, y in zip(cds, cis):
                    iv = jnp.minimum(iv, jnp.where(x == msv, y, _INT_MAX))
                isv = jnp.full((16,), jnp.min(iv))
                out_i = jnp.where(iota16 == it, isv, out_i)
                new_cds = tuple(
                    jnp.where((x == msv) & (y == isv), _F32_INF, x)
                    for x, y in zip(cds, cis))
                return (out_i,) + new_cds + cis
            init = (jnp.full((16,), 0, jnp.int32),
                    cand_d[pl.ds(0, 16)], cand_d[pl.ds(16, 16)],
                    cand_d[pl.ds(32, 16)], cand_d[pl.ds(48, 16)],
                    cand_i[pl.ds(0, 16)], cand_i[pl.ds(16, 16)],
                    cand_i[pl.ds(32, 16)], cand_i[pl.ds(48, 16)])
            out_i = lax.fori_loop(0, _TOPK, p3, init)[0]
            gbuf[c, pl.ds(r0, _TOPK)] = out_i
            return 0
        lax.fori_loop(0, qpw, per_query, 0)

        for c in range(nch):
            cf = pltpu.async_copy(feats_hbm.at[gbuf.at[c]], rows_f, s1)
            ck = pltpu.async_copy(keys_hbm.at[gbuf.at[c]], rows_k, s2)
            cf.wait()
            ck.wait()
            sf = pltpu.async_copy(rows_f, out_f.at[sbuf.at[c]], s1)
            sk = pltpu.async_copy(rows_k, out_k.at[sbuf.at[c]], s2)
            sf.wait()
            sk.wait()

    return body(gsel, d2flat, key_feats, keys)


def _head_body(fj_ref, kn_ref, q_ref, wp_ref, w_ref, o_ref):
    j = pl.program_id(1)
    dp = q_ref[...] - kn_ref[...]
    pe = jnp.dot(dp, wp_ref[...], preferred_element_type=jnp.float32)
    f = jnp.maximum(
        jnp.dot(fj_ref[...] + pe, w_ref[...], preferred_element_type=jnp.float32),
        0.0)

    @pl.when(j == 0)
    def _():
        o_ref[...] = f

    @pl.when(j > 0)
    def _():
        o_ref[...] = jnp.maximum(o_ref[...], f)


def _head(fj_nm, kn_nm, queries, w_pos, w):
    q_n, d_n = queries.shape
    h_n = w.shape[1]
    qb = 128
    nqb = q_n // qb
    return pl.pallas_call(
        _head_body,
        grid=(nqb, _TOPK),
        in_specs=[
            pl.BlockSpec((qb, d_n), lambda i, j: (j * nqb + i, 0)),
            pl.BlockSpec((qb, d_n), lambda i, j: (j * nqb + i, 0)),
            pl.BlockSpec((qb, d_n), lambda i, j: (i, 0)),
            pl.BlockSpec((d_n, d_n), lambda i, j: (0, 0)),
            pl.BlockSpec((d_n, h_n), lambda i, j: (0, 0)),
        ],
        out_specs=pl.BlockSpec((qb, h_n), lambda i, j: (i, 0)),
        out_shape=jax.ShapeDtypeStruct((q_n, h_n), jnp.float32),
    )(fj_nm, kn_nm, queries, w_pos, w)


def kernel(queries, keys, key_feats, W_pos, W, k):
    q_n, d_n = queries.shape
    k_n = keys.shape[0]
    nkb = -(-k_n // _KB)
    kp = nkb * _KB

    d2flat, gm3 = _dist_tiles(queries, keys, k_n)
    ng = kp // _GS
    gmins = jnp.transpose(gm3, (1, 0, 2)).reshape(q_n, ng)
    ngp = -(-ng // 128) * 128
    gmins = jnp.pad(gmins, ((0, 0), (0, ngp - ng)), constant_values=jnp.inf)
    gsel = _select_groups(gmins, ng)

    fj_nm, kn_nm = _sc_select_gather(gsel, d2flat, key_feats, keys, ng, q_n)

    return _head(fj_nm, kn_nm, queries, W_pos, W)


# speedup vs baseline: 2.0841x; 1.0108x over previous
import functools

import jax
import jax.numpy as jnp
import numpy as np
from jax import lax
from jax.experimental import pallas as pl
from jax.experimental.pallas import tpu as pltpu
from jax.experimental.pallas import tpu_sc as plsc

_TOPK = 16
_KB = 4096
_GS = 64
_CAP = 64
_INT_MAX = 2**31 - 1
_F32_INF = np.float32(np.inf)


def _dist_tile_body(q_ref, k_ref, d_out, g_out, *, kb, k_real):
    q = q_ref[...]
    kblk = k_ref[...]
    qk = lax.dot_general(q, kblk, (((1,), (1,)), ((), ())),
                         preferred_element_type=jnp.float32)
    q_sq = jnp.sum(q * q, axis=1, keepdims=True)
    k_sq = jnp.sum(kblk * kblk, axis=1)
    d2 = q_sq + k_sq[None, :] - 2.0 * qk
    col0 = pl.program_id(0) * kb
    cols = col0 + lax.broadcasted_iota(jnp.int32, (1, kb), 1)
    d2 = jnp.where(cols >= k_real, _F32_INF, d2)
    d_out[...] = jnp.concatenate(
        [d2[:, s * 128:(s + 1) * 128] for s in range(kb // 128)], axis=0)
    gm = jnp.concatenate(
        [jnp.min(d2[:, g * _GS:(g + 1) * _GS], axis=1, keepdims=True)
         for g in range(kb // _GS)], axis=1)
    g_out[...] = gm[None]


def _dist_tiles(queries, keys, k_real):
    q_n, d_n = queries.shape
    nkb = -(-k_real // _KB)
    kp = nkb * _KB
    ng = _KB // _GS
    body = functools.partial(_dist_tile_body, kb=_KB, k_real=k_real)
    return pl.pallas_call(
        body,
        grid=(nkb,),
        in_specs=[
            pl.BlockSpec((q_n, d_n), lambda i: (0, 0)),
            pl.BlockSpec((_KB, d_n), lambda i: (i, 0)),
        ],
        out_specs=[
            pl.BlockSpec((q_n * (_KB // 128), 128), lambda i: (i, 0)),
            pl.BlockSpec((1, q_n, ng), lambda i: (i, 0, 0)),
        ],
        out_shape=[
            jax.ShapeDtypeStruct((q_n * (kp // 128), 128), jnp.float32),
            jax.ShapeDtypeStruct((nkb, q_n, ng), jnp.float32),
        ],
    )(queries, keys)


def _gsel_body(g_ref, o_ref, *, topk, ng_real):
    d = g_ref[...]
    w = d.shape[1]
    cols = lax.broadcasted_iota(jnp.int32, (1, w), 1)
    d = jnp.where(cols >= ng_real, _F32_INF, d)
    outs = []
    for _ in range(topk):
        m = jnp.min(d, axis=1, keepdims=True)
        eq = d == m
        sel = jnp.min(jnp.where(eq, cols, _INT_MAX), axis=1, keepdims=True)
        outs.append(sel)
        d = jnp.where(cols == sel, _F32_INF, d)
    o_ref[...] = jnp.concatenate(outs, axis=1)


def _select_groups(gmins, ng_real):
    q_n, ngp = gmins.shape
    body = functools.partial(_gsel_body, topk=_TOPK, ng_real=ng_real)
    return pl.pallas_call(
        body,
        in_specs=[pl.BlockSpec((q_n, ngp), lambda: (0, 0))],
        out_specs=pl.BlockSpec((q_n, _TOPK), lambda: (0, 0)),
        out_shape=jax.ShapeDtypeStruct((q_n, _TOPK), jnp.int32),
    )(gmins)


def _lane_bcast(v, j):
    return v.at[jnp.full((16,), j, jnp.int32)].get(mode="promise_in_bounds")


def _sc_select_gather(gsel, d2flat, key_feats, keys, ng_total, q_n):
    info = plsc.get_sparse_core_info()
    nc = info.num_cores
    nw = nc * info.num_subcores
    d_n = key_feats.shape[1]
    qpw = q_n // nw
    spg = _GS // 16
    nch = (qpw * _TOPK) // 128
    qpc = 128 // _TOPK
    mesh = plsc.VectorSubcoreMesh(core_axis_name="c", subcore_axis_name="s")

    @functools.partial(
        pl.kernel,
        mesh=mesh,
        out_type=[
            jax.ShapeDtypeStruct((q_n * _TOPK, d_n), jnp.float32),
            jax.ShapeDtypeStruct((q_n * _TOPK, d_n), jnp.float32),
        ],
        scratch_types=[
            pltpu.VMEM((qpw, _TOPK), jnp.int32),
            pltpu.VMEM((nch, 128), jnp.int32),
            pltpu.VMEM((nch, 128, 128), jnp.float32),
            pltpu.VMEM((nch, 128), jnp.int32),
            pltpu.VMEM((nch, 128), jnp.int32),
            pltpu.VMEM((_CAP,), jnp.float32),
            pltpu.VMEM((_CAP,), jnp.int32),
            pltpu.VMEM((128, d_n), jnp.float32),
            pltpu.VMEM((128, d_n), jnp.float32),
            pltpu.SemaphoreType.DMA,
            pltpu.SemaphoreType.DMA,
        ],
        compiler_params=pltpu.CompilerParams(use_tc_tiling_on_sc=False,
                                             needs_layout_passes=False),
    )
    def body(gsel_hbm, d2_hbm, feats_hbm, keys_hbm, out_f, out_k,
             gsel_v, fidx, dbuf, gbuf, sbuf, cand_d, cand_i, rows_f, rows_k,
             s1, s2):
        wid = lax.axis_index("s") * nc + lax.axis_index("c")
        qbase = wid * qpw
        iota16 = lax.iota(jnp.int32, 16)

        pltpu.sync_copy(gsel_hbm.at[pl.ds(qbase, qpw)], gsel_v)

        def build(ql, _):
            c = ql // qpc
            r0 = (ql % qpc) * _TOPK
            g16 = gsel_v[ql]
            fidx[c, pl.ds(r0, _TOPK)] = (g16 >> 1) * q_n + (qbase + ql)
            sbuf[c, pl.ds(r0, _TOPK)] = iota16 * q_n + (qbase + ql)
            return 0
        lax.fori_loop(0, qpw, build, 0)

        copies = [pltpu.async_copy(d2_hbm.at[fidx.at[c]], dbuf.at[c], s1)
                  for c in range(nch)]
        for cp in copies:
            cp.wait()

        def per_query(ql, _):
            c = ql // qpc
            r0 = (ql % qpc) * _TOPK
            g16 = gsel_v[ql]

            def p1(t, r):
                j = t // spg
                h = jnp.max(_lane_bcast(g16, j)) & 1
                v = dbuf[c, r0 + j, pl.ds(h * _GS + (t % spg) * 16, 16)]
                sv = lax.sort(v)
                return lax.sort(jnp.minimum(r, lax.rev(sv, (0,))))
            r = lax.fori_loop(0, _TOPK * spg, p1, jnp.full((16,), _F32_INF))
            v16 = _lane_bcast(r, 15)

            for b in range(_CAP // 16):
                cand_d[pl.ds(b * 16, 16)] = jnp.full((16,), _F32_INF)
                cand_i[pl.ds(b * 16, 16)] = jnp.full((16,), _INT_MAX, jnp.int32)

            def p2(t, cnt):
                j = t // spg
                s = t % spg
                gj = _lane_bcast(g16, j)
                h = jnp.max(gj) & 1
                v = dbuf[c, r0 + j, pl.ds(h * _GS + s * 16, 16)]
                iv = gj * _GS + s * 16 + iota16
                m = (v <= v16) & (cnt < _CAP - 16)
                plsc.store_compressed(cand_d.at[pl.ds(cnt, 16)], v, mask=m)
                plsc.store_compressed(cand_i.at[pl.ds(cnt, 16)], iv, mask=m)
                npick = jnp.max(plsc.all_reduce_population_count(m))
                return cnt + npick
            lax.fori_loop(0, _TOPK * spg, p2, jnp.int32(0))

            def p3(it, carry):
                out_i, cd0, cd1, cd2, cd3, ci0, ci1, ci2, ci3 = carry
                cds = (cd0, cd1, cd2, cd3)
                cis = (ci0, ci1, ci2, ci3)
                mv = cds[0]
                for x in cds[1:]:
                    mv = jnp.minimum(mv, x)
                ms = jnp.min(mv)
                msv = jnp.full((16,), ms)
                iv = jnp.full((16,), _INT_MAX, jnp.int32)
                for x, y in zip(cds, cis):
                    iv = jnp.minimum(iv, jnp.where(x == msv, y, _INT_MAX))
                isv = jnp.full((16,), jnp.min(iv))
                out_i = jnp.where(iota16 == it, isv, out_i)
                new_cds = tuple(
                    jnp.where((x == msv) & (y == isv), _F32_INF, x)
                    for x, y in zip(cds, cis))
                return (out_i,) + new_cds + cis
            init = (jnp.full((16,), 0, jnp.int32),
                    cand_d[pl.ds(0, 16)], cand_d[pl.ds(16, 16)],
                    cand_d[pl.ds(32, 16)], cand_d[pl.ds(48, 16)],
                    cand_i[pl.ds(0, 16)], cand_i[pl.ds(16, 16)],
                    cand_i[pl.ds(32, 16)], cand_i[pl.ds(48, 16)])
            out_i = lax.fori_loop(0, _TOPK, p3, init)[0]
            gbuf[c, pl.ds(r0, _TOPK)] = out_i
            return 0
        lax.fori_loop(0, qpw, per_query, 0)

        for c in range(nch):
            cf = pltpu.async_copy(feats_hbm.at[gbuf.at[c]], rows_f, s1)
            ck = pltpu.async_copy(keys_hbm.at[gbuf.at[c]], rows_k, s2)
            cf.wait()
            ck.wait()
            sf = pltpu.async_copy(rows_f, out_f.at[sbuf.at[c]], s1)
            sk = pltpu.async_copy(rows_k, out_k.at[sbuf.at[c]], s2)
            sf.wait()
            sk.wait()

    return body(gsel, d2flat, key_feats, keys)


def _head_body(fj_ref, kn_ref, q_ref, wp_ref, w_ref, o_ref):
    j = pl.program_id(1)
    dp = q_ref[...] - kn_ref[...]
    pe = jnp.dot(dp, wp_ref[...], preferred_element_type=jnp.float32)
    f = jnp.maximum(
        jnp.dot(fj_ref[...] + pe, w_ref[...], preferred_element_type=jnp.float32),
        0.0)

    @pl.when(j == 0)
    def _():
        o_ref[...] = f

    @pl.when(j > 0)
    def _():
        o_ref[...] = jnp.maximum(o_ref[...], f)


def _head(fj_nm, kn_nm, queries, w_pos, w):
    q_n, d_n = queries.shape
    h_n = w.shape[1]
    qb = 128
    nqb = q_n // qb
    return pl.pallas_call(
        _head_body,
        grid=(nqb, _TOPK),
        in_specs=[
            pl.BlockSpec((qb, d_n), lambda i, j: (j * nqb + i, 0)),
            pl.BlockSpec((qb, d_n), lambda i, j: (j * nqb + i, 0)),
            pl.BlockSpec((qb, d_n), lambda i, j: (i, 0)),
            pl.BlockSpec((d_n, d_n), lambda i, j: (0, 0)),
            pl.BlockSpec((d_n, h_n), lambda i, j: (0, 0)),
        ],
        out_specs=pl.BlockSpec((qb, h_n), lambda i, j: (i, 0)),
        out_shape=jax.ShapeDtypeStruct((q_n, h_n), jnp.float32),
    )(fj_nm, kn_nm, queries, w_pos, w)


def kernel(queries, keys, key_feats, W_pos, W, k):
    q_n, d_n = queries.shape
    k_n = keys.shape[0]
    nkb = -(-k_n // _KB)
    kp = nkb * _KB

    d2flat, gm3 = _dist_tiles(queries, keys, k_n)
    ng = kp // _GS
    gmins = jnp.transpose(gm3, (1, 0, 2)).reshape(q_n, ng)
    ngp = -(-ng // 128) * 128
    gmins = jnp.pad(gmins, ((0, 0), (0, ngp - ng)), constant_values=jnp.inf)
    gsel = _select_groups(gmins, ng)

    fj_nm, kn_nm = _sc_select_gather(gsel, d2flat, key_feats, keys, ng, q_n)

    return _head(fj_nm, kn_nm, queries, W_pos, W)


# speedup vs baseline: 2.3734x; 1.1388x over previous
import functools

import jax
import jax.numpy as jnp
import numpy as np
from jax import lax
from jax.experimental import pallas as pl
from jax.experimental.pallas import tpu as pltpu
from jax.experimental.pallas import tpu_sc as plsc

_TOPK = 16
_KB = 4096
_GS = 64
_CAP = 64
_INT_MAX = 2**31 - 1
_F32_INF = np.float32(np.inf)


def _dist_tile_body(q_ref, k_ref, d_out, g_out, *, kb, k_real):
    q = q_ref[...]
    kblk = k_ref[...]
    qk = lax.dot_general(q, kblk, (((1,), (1,)), ((), ())),
                         preferred_element_type=jnp.float32)
    q_sq = jnp.sum(q * q, axis=1, keepdims=True)
    k_sq = jnp.sum(kblk * kblk, axis=1)
    d2 = q_sq + k_sq[None, :] - 2.0 * qk
    col0 = pl.program_id(0) * kb
    cols = col0 + lax.broadcasted_iota(jnp.int32, (1, kb), 1)
    d2 = jnp.where(cols >= k_real, _F32_INF, d2)
    d_out[...] = jnp.concatenate(
        [d2[:, s * 128:(s + 1) * 128] for s in range(kb // 128)], axis=0)
    g_out[...] = jnp.concatenate(
        [jnp.min(d2[:, g * _GS:(g + 1) * _GS], axis=1, keepdims=True)
         for g in range(kb // _GS)], axis=1)[None]


def _dist_tiles(queries, keys, k_real):
    q_n, d_n = queries.shape
    nkb = -(-k_real // _KB)
    kp = nkb * _KB
    ng = _KB // _GS
    body = functools.partial(_dist_tile_body, kb=_KB, k_real=k_real)
    return pl.pallas_call(
        body,
        grid=(nkb,),
        in_specs=[
            pl.BlockSpec((q_n, d_n), lambda i: (0, 0)),
            pl.BlockSpec((_KB, d_n), lambda i: (i, 0)),
        ],
        out_specs=[
            pl.BlockSpec((q_n * (_KB // 128), 128), lambda i: (i, 0)),
            pl.BlockSpec((1, q_n, ng), lambda i: (i, 0, 0)),
        ],
        out_shape=[
            jax.ShapeDtypeStruct((q_n * (kp // 128), 128), jnp.float32),
            jax.ShapeDtypeStruct((nkb, q_n, ng), jnp.float32),
        ],
    )(queries, keys)


def _gsel_body(g_ref, o_ref, *, topk, ng_real):
    nkb, q_n, ngt = g_ref.shape
    pad = -(-(nkb * ngt) // 128) * 128 - nkb * ngt
    parts = [g_ref[t] for t in range(nkb)]
    if pad:
        parts.append(jnp.full((q_n, pad), _F32_INF))
    d = jnp.concatenate(parts, axis=1)
    w = d.shape[1]
    cols = lax.broadcasted_iota(jnp.int32, (1, w), 1)
    d = jnp.where(cols >= ng_real, _F32_INF, d)
    outs = []
    for _ in range(topk):
        m = jnp.min(d, axis=1, keepdims=True)
        eq = d == m
        sel = jnp.min(jnp.where(eq, cols, _INT_MAX), axis=1, keepdims=True)
        outs.append(sel)
        d = jnp.where(cols == sel, _F32_INF, d)
    o_ref[...] = jnp.concatenate(outs, axis=1)


def _select_groups(gm3, ng_real):
    nkb, q_n, ngt = gm3.shape
    body = functools.partial(_gsel_body, topk=_TOPK, ng_real=ng_real)
    return pl.pallas_call(
        body,
        in_specs=[pl.BlockSpec((nkb, q_n, ngt), lambda: (0, 0, 0))],
        out_specs=pl.BlockSpec((q_n, _TOPK), lambda: (0, 0)),
        out_shape=jax.ShapeDtypeStruct((q_n, _TOPK), jnp.int32),
    )(gm3)


def _lane_bcast(v, j):
    return v.at[jnp.full((16,), j, jnp.int32)].get(mode="promise_in_bounds")


def _sc_select_gather(gsel, d2flat, key_feats, keys, ng_total, q_n):
    info = plsc.get_sparse_core_info()
    nc = info.num_cores
    nw = nc * info.num_subcores
    d_n = key_feats.shape[1]
    qpw = q_n // nw
    spg = _GS // 16
    nch = (qpw * _TOPK) // 128
    qpc = 128 // _TOPK
    mesh = plsc.VectorSubcoreMesh(core_axis_name="c", subcore_axis_name="s")

    @functools.partial(
        pl.kernel,
        mesh=mesh,
        out_type=[
            jax.ShapeDtypeStruct((q_n * _TOPK, d_n), jnp.float32),
            jax.ShapeDtypeStruct((q_n * _TOPK, d_n), jnp.float32),
        ],
        scratch_types=[
            pltpu.VMEM((qpw, _TOPK), jnp.int32),
            pltpu.VMEM((nch, 128), jnp.int32),
            pltpu.VMEM((nch, 128, 128), jnp.float32),
            pltpu.VMEM((nch, 128), jnp.int32),
            pltpu.VMEM((nch, 128), jnp.int32),
            pltpu.VMEM((_CAP,), jnp.float32),
            pltpu.VMEM((_CAP,), jnp.int32),
            pltpu.VMEM((128, d_n), jnp.float32),
            pltpu.VMEM((128, d_n), jnp.float32),
            pltpu.SemaphoreType.DMA,
            pltpu.SemaphoreType.DMA,
        ],
        compiler_params=pltpu.CompilerParams(use_tc_tiling_on_sc=False,
                                             needs_layout_passes=False),
    )
    def body(gsel_hbm, d2_hbm, feats_hbm, keys_hbm, out_f, out_k,
             gsel_v, fidx, dbuf, gbuf, sbuf, cand_d, cand_i, rows_f, rows_k,
             s1, s2):
        wid = lax.axis_index("s") * nc + lax.axis_index("c")
        qbase = wid * qpw
        iota16 = lax.iota(jnp.int32, 16)

        pltpu.sync_copy(gsel_hbm.at[pl.ds(qbase, qpw)], gsel_v)

        def build(ql, _):
            c = ql // qpc
            r0 = (ql % qpc) * _TOPK
            g16 = gsel_v[ql]
            fidx[c, pl.ds(r0, _TOPK)] = (g16 >> 1) * q_n + (qbase + ql)
            sbuf[c, pl.ds(r0, _TOPK)] = iota16 * q_n + (qbase + ql)
            return 0
        lax.fori_loop(0, qpw, build, 0)

        copies = [pltpu.async_copy(d2_hbm.at[fidx.at[c]], dbuf.at[c], s1)
                  for c in range(nch)]
        for cp in copies:
            cp.wait()

        def per_query(ql, _):
            c = ql // qpc
            r0 = (ql % qpc) * _TOPK
            g16 = gsel_v[ql]

            def p1(t, r):
                j = t // spg
                h = jnp.max(_lane_bcast(g16, j)) & 1
                v = dbuf[c, r0 + j, pl.ds(h * _GS + (t % spg) * 16, 16)]
                sv = lax.sort(v)
                return lax.sort(jnp.minimum(r, lax.rev(sv, (0,))))
            r = lax.fori_loop(0, _TOPK * spg, p1, jnp.full((16,), _F32_INF))
            v16 = _lane_bcast(r, 15)

            for b in range(_CAP // 16):
                cand_d[pl.ds(b * 16, 16)] = jnp.full((16,), _F32_INF)
                cand_i[pl.ds(b * 16, 16)] = jnp.full((16,), _INT_MAX, jnp.int32)

            def p2(t, cnt):
                j = t // spg
                s = t % spg
                gj = _lane_bcast(g16, j)
                h = jnp.max(gj) & 1
                v = dbuf[c, r0 + j, pl.ds(h * _GS + s * 16, 16)]
                iv = gj * _GS + s * 16 + iota16
                m = (v <= v16) & (cnt < _CAP - 16)
                plsc.store_compressed(cand_d.at[pl.ds(cnt, 16)], v, mask=m)
                plsc.store_compressed(cand_i.at[pl.ds(cnt, 16)], iv, mask=m)
                npick = jnp.max(plsc.all_reduce_population_count(m))
                return cnt + npick
            lax.fori_loop(0, _TOPK * spg, p2, jnp.int32(0))

            def p3(it, carry):
                out_i, cd0, cd1, cd2, cd3, ci0, ci1, ci2, ci3 = carry
                cds = (cd0, cd1, cd2, cd3)
                cis = (ci0, ci1, ci2, ci3)
                mv = cds[0]
                for x in cds[1:]:
                    mv = jnp.minimum(mv, x)
                ms = jnp.min(mv)
                msv = jnp.full((16,), ms)
                iv = jnp.full((16,), _INT_MAX, jnp.int32)
                for x, y in zip(cds, cis):
                    iv = jnp.minimum(iv, jnp.where(x == msv, y, _INT_MAX))
                isv = jnp.full((16,), jnp.min(iv))
                out_i = jnp.where(iota16 == it, isv, out_i)
                new_cds = tuple(
                    jnp.where((x == msv) & (y == isv), _F32_INF, x)
                    for x, y in zip(cds, cis))
                return (out_i,) + new_cds + cis
            init = (jnp.full((16,), 0, jnp.int32),
                    cand_d[pl.ds(0, 16)], cand_d[pl.ds(16, 16)],
                    cand_d[pl.ds(32, 16)], cand_d[pl.ds(48, 16)],
                    cand_i[pl.ds(0, 16)], cand_i[pl.ds(16, 16)],
                    cand_i[pl.ds(32, 16)], cand_i[pl.ds(48, 16)])
            out_i = lax.fori_loop(0, _TOPK, p3, init)[0]
            gbuf[c, pl.ds(r0, _TOPK)] = out_i
            return 0
        lax.fori_loop(0, qpw, per_query, 0)

        for c in range(nch):
            cf = pltpu.async_copy(feats_hbm.at[gbuf.at[c]], rows_f, s1)
            ck = pltpu.async_copy(keys_hbm.at[gbuf.at[c]], rows_k, s2)
            cf.wait()
            ck.wait()
            sf = pltpu.async_copy(rows_f, out_f.at[sbuf.at[c]], s1)
            sk = pltpu.async_copy(rows_k, out_k.at[sbuf.at[c]], s2)
            sf.wait()
            sk.wait()

    return body(gsel, d2flat, key_feats, keys)


def _head_body(fj_ref, kn_ref, q_ref, wp_ref, w_ref, o_ref, *, jpb, q_n):
    step = pl.program_id(0)
    q = q_ref[...]
    wp = wp_ref[...]
    w = w_ref[...]
    acc = None
    for jj in range(jpb):
        kn = kn_ref[pl.ds(jj * q_n, q_n), :]
        fj = fj_ref[pl.ds(jj * q_n, q_n), :]
        pe = jnp.dot(q - kn, wp, preferred_element_type=jnp.float32)
        f = jnp.maximum(
            jnp.dot(fj + pe, w, preferred_element_type=jnp.float32), 0.0)
        acc = f if acc is None else jnp.maximum(acc, f)

    @pl.when(step == 0)
    def _():
        o_ref[...] = acc

    @pl.when(step > 0)
    def _():
        o_ref[...] = jnp.maximum(o_ref[...], acc)


def _head(fj_nm, kn_nm, queries, w_pos, w):
    q_n, d_n = queries.shape
    h_n = w.shape[1]
    jpb = 4
    body = functools.partial(_head_body, jpb=jpb, q_n=q_n)
    return pl.pallas_call(
        body,
        grid=(_TOPK // jpb,),
        in_specs=[
            pl.BlockSpec((jpb * q_n, d_n), lambda i: (i, 0)),
            pl.BlockSpec((jpb * q_n, d_n), lambda i: (i, 0)),
            pl.BlockSpec((q_n, d_n), lambda i: (0, 0)),
            pl.BlockSpec((d_n, d_n), lambda i: (0, 0)),
            pl.BlockSpec((d_n, h_n), lambda i: (0, 0)),
        ],
        out_specs=pl.BlockSpec((q_n, h_n), lambda i: (0, 0)),
        out_shape=jax.ShapeDtypeStruct((q_n, h_n), jnp.float32),
    )(fj_nm, kn_nm, queries, w_pos, w)


def kernel(queries, keys, key_feats, W_pos, W, k):
    q_n, d_n = queries.shape
    k_n = keys.shape[0]
    nkb = -(-k_n // _KB)
    kp = nkb * _KB

    d2flat, gm3 = _dist_tiles(queries, keys, k_n)
    ng = kp // _GS
    gsel = _select_groups(gm3, ng)

    fj_nm, kn_nm = _sc_select_gather(gsel, d2flat, key_feats, keys, ng, q_n)

    return _head(fj_nm, kn_nm, queries, W_pos, W)


# speedup vs baseline: 2.6062x; 1.0981x over previous
import functools

import jax
import jax.numpy as jnp
import numpy as np
from jax import lax
from jax.experimental import pallas as pl
from jax.experimental.pallas import tpu as pltpu
from jax.experimental.pallas import tpu_sc as plsc

_TOPK = 16
_KB = 4096
_GS = 64
_CAP = 64
_INT_MAX = 2**31 - 1
_F32_INF = np.float32(np.inf)


def _dist_tile_body(q_ref, k_ref, d_out, g_out, *, kb, k_real):
    q = q_ref[...]
    kblk = k_ref[...]
    qk = lax.dot_general(q, kblk, (((1,), (1,)), ((), ())),
                         preferred_element_type=jnp.float32)
    q_sq = jnp.sum(q * q, axis=1, keepdims=True)
    k_sq = jnp.sum(kblk * kblk, axis=1)
    d2 = q_sq + k_sq[None, :] - 2.0 * qk
    col0 = pl.program_id(0) * kb
    cols = col0 + lax.broadcasted_iota(jnp.int32, (1, kb), 1)
    d2 = jnp.where(cols >= k_real, _F32_INF, d2)
    d_out[...] = jnp.concatenate(
        [d2[:, s * 128:(s + 1) * 128] for s in range(kb // 128)], axis=0)
    g_out[...] = jnp.concatenate(
        [jnp.min(d2[:, g * _GS:(g + 1) * _GS], axis=1, keepdims=True)
         for g in range(kb // _GS)], axis=1)[None]


def _dist_tiles(queries, keys, k_real):
    q_n, d_n = queries.shape
    nkb = -(-k_real // _KB)
    kp = nkb * _KB
    ng = _KB // _GS
    body = functools.partial(_dist_tile_body, kb=_KB, k_real=k_real)
    return pl.pallas_call(
        body,
        grid=(nkb,),
        in_specs=[
            pl.BlockSpec((q_n, d_n), lambda i: (0, 0)),
            pl.BlockSpec((_KB, d_n), lambda i: (i, 0)),
        ],
        out_specs=[
            pl.BlockSpec((q_n * (_KB // 128), 128), lambda i: (i, 0)),
            pl.BlockSpec((1, q_n, ng), lambda i: (i, 0, 0)),
        ],
        out_shape=[
            jax.ShapeDtypeStruct((q_n * (kp // 128), 128), jnp.float32),
            jax.ShapeDtypeStruct((nkb, q_n, ng), jnp.float32),
        ],
    )(queries, keys)


def _gsel_body(g_ref, o_ref, *, topk, ng_real):
    nkb, q_n, ngt = g_ref.shape
    pad = -(-(nkb * ngt) // 128) * 128 - nkb * ngt
    parts = [g_ref[t] for t in range(nkb)]
    if pad:
        parts.append(jnp.full((q_n, pad), _F32_INF))
    d = jnp.concatenate(parts, axis=1)
    w = d.shape[1]
    cols = lax.broadcasted_iota(jnp.int32, (1, w), 1)
    d = jnp.where(cols >= ng_real, _F32_INF, d)
    outs = []
    for _ in range(topk):
        m = jnp.min(d, axis=1, keepdims=True)
        eq = d == m
        sel = jnp.min(jnp.where(eq, cols, _INT_MAX), axis=1, keepdims=True)
        outs.append(sel)
        d = jnp.where(cols == sel, _F32_INF, d)
    o_ref[...] = jnp.concatenate(outs, axis=1)


def _select_groups(gm3, ng_real):
    nkb, q_n, ngt = gm3.shape
    body = functools.partial(_gsel_body, topk=_TOPK, ng_real=ng_real)
    return pl.pallas_call(
        body,
        in_specs=[pl.BlockSpec((nkb, q_n, ngt), lambda: (0, 0, 0))],
        out_specs=pl.BlockSpec((q_n, _TOPK), lambda: (0, 0)),
        out_shape=jax.ShapeDtypeStruct((q_n, _TOPK), jnp.int32),
    )(gm3)


def _lane_bcast(v, j):
    return v.at[jnp.full((16,), j, jnp.int32)].get(mode="promise_in_bounds")


def _sc_select_gather(gsel, d2flat, table, ng_total, q_n):
    info = plsc.get_sparse_core_info()
    nc = info.num_cores
    nw = nc * info.num_subcores
    t_n = table.shape[1]
    qpw = q_n // nw
    spg = _GS // 16
    nch = (qpw * _TOPK) // 128
    qpc = 128 // _TOPK
    mesh = plsc.VectorSubcoreMesh(core_axis_name="c", subcore_axis_name="s")

    @functools.partial(
        pl.kernel,
        mesh=mesh,
        out_type=jax.ShapeDtypeStruct((q_n * _TOPK, t_n), jnp.float32),
        scratch_types=[
            pltpu.VMEM((qpw, _TOPK), jnp.int32),
            pltpu.VMEM((nch, 128), jnp.int32),
            pltpu.VMEM((nch, 128, 128), jnp.float32),
            pltpu.VMEM((nch, 128), jnp.int32),
            pltpu.VMEM((nch, 128), jnp.int32),
            pltpu.VMEM((_CAP,), jnp.float32),
            pltpu.VMEM((_CAP,), jnp.int32),
            pltpu.VMEM((2, 128, t_n), jnp.float32),
            pltpu.SemaphoreType.DMA,
            pltpu.SemaphoreType.DMA,
        ],
        compiler_params=pltpu.CompilerParams(use_tc_tiling_on_sc=False,
                                             needs_layout_passes=False),
    )
    def body(gsel_hbm, d2_hbm, table_hbm, out_c,
             gsel_v, fidx, dbuf, gbuf, sbuf, cand_d, cand_i, rows_c,
             s1, s2):
        wid = lax.axis_index("s") * nc + lax.axis_index("c")
        qbase = wid * qpw
        iota16 = lax.iota(jnp.int32, 16)

        pltpu.sync_copy(gsel_hbm.at[pl.ds(qbase, qpw)], gsel_v)

        def build(ql, _):
            c = ql // qpc
            r0 = (ql % qpc) * _TOPK
            g16 = gsel_v[ql]
            fidx[c, pl.ds(r0, _TOPK)] = (g16 >> 1) * q_n + (qbase + ql)
            sbuf[c, pl.ds(r0, _TOPK)] = iota16 * q_n + (qbase + ql)
            return 0
        lax.fori_loop(0, qpw, build, 0)

        copies = [pltpu.async_copy(d2_hbm.at[fidx.at[c]], dbuf.at[c], s1)
                  for c in range(nch)]

        def per_query(ql, c):
            r0 = (ql % qpc) * _TOPK
            g16 = gsel_v[ql]

            def p1(t, r):
                j = t // spg
                h = jnp.max(_lane_bcast(g16, j)) & 1
                v = dbuf[c, r0 + j, pl.ds(h * _GS + (t % spg) * 16, 16)]
                sv = lax.sort(v)
                return lax.sort(jnp.minimum(r, lax.rev(sv, (0,))))
            r = lax.fori_loop(0, _TOPK * spg, p1, jnp.full((16,), _F32_INF))
            v16 = _lane_bcast(r, 15)

            for b in range(_CAP // 16):
                cand_d[pl.ds(b * 16, 16)] = jnp.full((16,), _F32_INF)
                cand_i[pl.ds(b * 16, 16)] = jnp.full((16,), _INT_MAX, jnp.int32)

            def p2(t, cnt):
                j = t // spg
                s = t % spg
                gj = _lane_bcast(g16, j)
                h = jnp.max(gj) & 1
                v = dbuf[c, r0 + j, pl.ds(h * _GS + s * 16, 16)]
                iv = gj * _GS + s * 16 + iota16
                m = (v <= v16) & (cnt < _CAP - 16)
                plsc.store_compressed(cand_d.at[pl.ds(cnt, 16)], v, mask=m)
                plsc.store_compressed(cand_i.at[pl.ds(cnt, 16)], iv, mask=m)
                npick = jnp.max(plsc.all_reduce_population_count(m))
                return cnt + npick
            lax.fori_loop(0, _TOPK * spg, p2, jnp.int32(0))

            def p3(it, carry):
                out_i, cd0, cd1, cd2, cd3, ci0, ci1, ci2, ci3 = carry
                cds = (cd0, cd1, cd2, cd3)
                cis = (ci0, ci1, ci2, ci3)
                mv = cds[0]
                for x in cds[1:]:
                    mv = jnp.minimum(mv, x)
                ms = jnp.min(mv)
                msv = jnp.full((16,), ms)
                iv = jnp.full((16,), _INT_MAX, jnp.int32)
                for x, y in zip(cds, cis):
                    iv = jnp.minimum(iv, jnp.where(x == msv, y, _INT_MAX))
                isv = jnp.full((16,), jnp.min(iv))
                out_i = jnp.where(iota16 == it, isv, out_i)
                new_cds = tuple(
                    jnp.where((x == msv) & (y == isv), _F32_INF, x)
                    for x, y in zip(cds, cis))
                return (out_i,) + new_cds + cis
            init = (jnp.full((16,), 0, jnp.int32),
                    cand_d[pl.ds(0, 16)], cand_d[pl.ds(16, 16)],
                    cand_d[pl.ds(32, 16)], cand_d[pl.ds(48, 16)],
                    cand_i[pl.ds(0, 16)], cand_i[pl.ds(16, 16)],
                    cand_i[pl.ds(32, 16)], cand_i[pl.ds(48, 16)])
            out_i = lax.fori_loop(0, _TOPK, p3, init)[0]
            gbuf[c, pl.ds(r0, _TOPK)] = out_i
            return c

        scat = [None, None]
        for c in range(nch):
            copies[c].wait()
            lax.fori_loop(c * qpc, (c + 1) * qpc, per_query, c)
            if scat[c % 2] is not None:
                scat[c % 2].wait()
            gth = pltpu.async_copy(table_hbm.at[gbuf.at[c]], rows_c.at[c % 2], s2)
            gth.wait()
            scat[c % 2] = pltpu.async_copy(rows_c.at[c % 2],
                                           out_c.at[sbuf.at[c]], s2)
        for sc_cp in scat:
            if sc_cp is not None:
                sc_cp.wait()

    return body(gsel, d2flat, table)


def _head_body(nm_ref, q_ref, wp_ref, w_ref, o_ref, *, jpb, q_n, d_n):
    step = pl.program_id(0)
    q = q_ref[...]
    wp = wp_ref[...]
    w = w_ref[...]
    acc = None
    for jj in range(jpb):
        slab = nm_ref[pl.ds(jj * q_n, q_n), :]
        fj = slab[:, :d_n]
        kn = slab[:, d_n:]
        pe = jnp.dot(q - kn, wp, preferred_element_type=jnp.float32)
        f = jnp.maximum(
            jnp.dot(fj + pe, w, preferred_element_type=jnp.float32), 0.0)
        acc = f if acc is None else jnp.maximum(acc, f)

    @pl.when(step == 0)
    def _():
        o_ref[...] = acc

    @pl.when(step > 0)
    def _():
        o_ref[...] = jnp.maximum(o_ref[...], acc)


def _head(nm, queries, w_pos, w):
    q_n, d_n = queries.shape
    h_n = w.shape[1]
    jpb = 4
    body = functools.partial(_head_body, jpb=jpb, q_n=q_n, d_n=d_n)
    return pl.pallas_call(
        body,
        grid=(_TOPK // jpb,),
        in_specs=[
            pl.BlockSpec((jpb * q_n, 2 * d_n), lambda i: (i, 0)),
            pl.BlockSpec((q_n, d_n), lambda i: (0, 0)),
            pl.BlockSpec((d_n, d_n), lambda i: (0, 0)),
            pl.BlockSpec((d_n, h_n), lambda i: (0, 0)),
        ],
        out_specs=pl.BlockSpec((q_n, h_n), lambda i: (0, 0)),
        out_shape=jax.ShapeDtypeStruct((q_n, h_n), jnp.float32),
    )(nm, queries, w_pos, w)


def kernel(queries, keys, key_feats, W_pos, W, k):
    q_n, d_n = queries.shape
    k_n = keys.shape[0]
    nkb = -(-k_n // _KB)
    kp = nkb * _KB

    d2flat, gm3 = _dist_tiles(queries, keys, k_n)
    ng = kp // _GS
    gsel = _select_groups(gm3, ng)

    table = jnp.concatenate([key_feats, keys], axis=1)
    nm = _sc_select_gather(gsel, d2flat, table, ng, q_n)

    return _head(nm, queries, W_pos, W)
